# Initial kernel scaffold; baseline (speedup 1.0000x reference)
import jax
import jax.numpy as jnp
from jax.experimental import pallas as pl


def kernel(input, query, edge_index, edge_type, adj_size, boundary, mask_ix, degree_in, rel_W, rel_b, lin_W, lin_b, ln_g, ln_b):
    raise NotImplementedError("write your pallas kernel here")


# speedup vs baseline: 1.2178x; 1.2178x over previous
import functools

import jax
import jax.numpy as jnp
from jax.experimental import pallas as pl
from jax.experimental.pallas import tpu as pltpu

NUM_ENTS = 10000
BATCH = 2
NUM_REL = 64
IN_DIM = 128
OUT_DIM = 128
NSEG = BATCH * NUM_ENTS
BR = 160
NBLK = NSEG // BR


def _logsum_body(deg_ref, out_ref):
    out_ref[...] = jnp.sum(jnp.log(deg_ref[...])).reshape(1, 1)


def _tail_body(sum_ref, maxr_ref, minr_ref, sq_ref, bnd_ref, inp_ref,
               deg_ref, logsum_ref, wi_ref, wr_ref, lb_ref, lng_ref,
               lnb_ref, out_ref):
    degb = deg_ref[0, 0, :]
    deg2 = degb[:, None]
    bnd = bnd_ref[...]
    agg_sum = sum_ref[...]
    maxr = maxr_ref[...]
    minr = minr_ref[...]
    sq = sq_ref[...]

    maxv = jnp.maximum(jnp.where(jnp.isfinite(maxr), maxr, 0.0), bnd)
    minv = jnp.minimum(jnp.where(jnp.isfinite(minr), minr, 0.0), bnd)
    meanv = (agg_sum + bnd) / deg2
    stdv = jnp.maximum(sq + bnd * bnd, 1e-6) / deg2

    scale = jnp.log(degb) / (jnp.sum(logsum_ref[...]) / NSEG)
    inv = 1.0 / jnp.maximum(scale, 0.01)

    acc = jnp.dot(inp_ref[...], wi_ref[...], preferred_element_type=jnp.float32)
    a1 = jnp.zeros((BR, OUT_DIM), jnp.float32)
    a2 = jnp.zeros((BR, OUT_DIM), jnp.float32)
    a3 = jnp.zeros((BR, OUT_DIM), jnp.float32)
    feats = (meanv, maxv, minv, stdv)
    for f in range(4):
        a1 = a1 + jnp.dot(feats[f], wr_ref[3 * f], preferred_element_type=jnp.float32)
        a2 = a2 + jnp.dot(feats[f], wr_ref[3 * f + 1], preferred_element_type=jnp.float32)
        a3 = a3 + jnp.dot(feats[f], wr_ref[3 * f + 2], preferred_element_type=jnp.float32)
    h = acc + a1 + scale[:, None] * a2 + inv[:, None] * a3 + lb_ref[...]
    mu = jnp.mean(h, axis=-1, keepdims=True)
    var = jnp.mean((h - mu) ** 2, axis=-1, keepdims=True)
    h = (h - mu) / jnp.sqrt(var + 1e-5) * lng_ref[...] + lnb_ref[...]
    out_ref[...] = jnp.maximum(h, 0.0)


@jax.jit
def _dense_tail(agg_sum, agg_max_raw, agg_min_raw, sq_sum, boundary, input,
                deg, wi, wr, lin_b, ln_g, ln_b):
    logsum = pl.pallas_call(
        _logsum_body,
        out_shape=jax.ShapeDtypeStruct((1, 1), jnp.float32),
        in_specs=[pl.BlockSpec((NSEG,), lambda: (0,))],
        out_specs=pl.BlockSpec((1, 1), lambda: (0, 0)),
    )(deg)

    blk = lambda: pl.BlockSpec((BR, IN_DIM), lambda i: (i, 0))
    full = lambda shape: pl.BlockSpec(shape, lambda i: tuple(0 for _ in shape))
    return pl.pallas_call(
        _tail_body,
        grid=(NBLK,),
        out_shape=jax.ShapeDtypeStruct((NSEG, OUT_DIM), jnp.float32),
        in_specs=[blk(), blk(), blk(), blk(), blk(), blk(),
                  pl.BlockSpec((1, 1, BR), lambda i: (i, 0, 0)), full((1, 1)),
                  full((IN_DIM, OUT_DIM)), full((12, IN_DIM, OUT_DIM)),
                  full((1, OUT_DIM)), full((1, OUT_DIM)), full((1, OUT_DIM))],
        out_specs=pl.BlockSpec((BR, OUT_DIM), lambda i: (i, 0)),
    )(agg_sum, agg_max_raw, agg_min_raw, sq_sum, boundary, input,
      deg.reshape(NBLK, 1, BR), logsum, wi, wr, lin_b, ln_g, ln_b)


def kernel(input, query, edge_index, edge_type, adj_size, boundary, mask_ix,
           degree_in, rel_W, rel_b, lin_W, lin_b, ln_g, ln_b):
    batch_size = query.shape[0]
    mask_batch = mask_ix[0]
    midx = mask_ix[1]
    all_rel = jnp.take(edge_type, midx, axis=0)
    all_head = jnp.take(edge_index[0], midx, axis=0)
    all_tail = jnp.take(edge_index[1], midx, axis=0)
    off_head = all_head + NUM_ENTS * mask_batch
    off_tail = all_tail + NUM_ENTS * mask_batch

    relation = (query @ rel_W + rel_b).reshape(batch_size * NUM_REL, IN_DIM)
    off_rel = all_rel + NUM_REL * mask_batch
    msgs = jnp.take(input, off_head, axis=0) * jnp.take(relation, off_rel, axis=0)

    degree_agg = jnp.zeros((input.shape[0],), jnp.float32).at[off_tail].set(
        jnp.take(degree_in, all_tail, axis=0))
    agg_sum = jax.ops.segment_sum(msgs, off_tail, num_segments=NSEG)
    agg_max = jax.ops.segment_max(msgs, off_tail, num_segments=NSEG)
    agg_min = jax.ops.segment_min(msgs, off_tail, num_segments=NSEG)
    sq_sum = jax.ops.segment_sum(msgs ** 2, off_tail, num_segments=NSEG)
    deg = degree_agg + 1.0

    wi = lin_W[:IN_DIM]
    wr = jnp.transpose(lin_W[IN_DIM:].reshape(IN_DIM, 4, 3, OUT_DIM),
                       (1, 2, 0, 3)).reshape(12, IN_DIM, OUT_DIM)
    return _dense_tail(agg_sum, agg_max, agg_min, sq_sum, boundary, input,
                       deg, wi, wr, lin_b.reshape(1, -1),
                       ln_g.reshape(1, -1), ln_b.reshape(1, -1))
